# baseline (speedup 1.0000x reference)
import functools

import jax
import jax.numpy as jnp
from jax import lax
from jax.experimental import pallas as pl
from jax.experimental.pallas import tpu as pltpu
from jax.experimental.pallas import tpu_sc as plsc

_N_EMBED = 8192
_EMBED_DIM = 32
_T = 512
_NT = 8192 // _T
_CHUNK = 2048
_ROWS = _N_EMBED // 8


def _argmin_body(flat_ref, sumf_ref, sume_ref, emb_ref, idx_ref, mt_ref):
    f = flat_ref[...]
    m2 = lax.dot_general(emb_ref[...] * 2.0, f, (((1,), (1,)), ((), ())),
                         preferred_element_type=jnp.float32)
    mt_ref[...] = m2.reshape(_ROWS, 8, _T)

    sumf = sumf_ref[...]
    iota0 = lax.broadcasted_iota(jnp.int32, (8, _T), 0)
    big = jnp.int32(2**31 - 1)

    acc_v = jnp.full((1, _T), jnp.inf, jnp.float32)
    acc_i = jnp.zeros((1, _T), jnp.int32)
    rows_per_chunk = _CHUNK // 8
    for c in range(_N_EMBED // _CHUNK):
        def body(r, carry, c=c):
            cv, ci = carry
            rg = c * rows_per_chunk + r
            mrow = mt_ref[rg]
            d = (sumf + sume_ref[rg]) - mrow
            lt = d < cv
            cv = jnp.where(lt, d, cv)
            ci = jnp.where(lt, iota0 + rg * 8, ci)
            return cv, ci
        cv, ci = lax.fori_loop(
            0, rows_per_chunk, body,
            (jnp.full((8, _T), jnp.inf, jnp.float32),
             jnp.zeros((8, _T), jnp.int32)))
        mc = jnp.min(cv, axis=0, keepdims=True)
        ixc = jnp.min(jnp.where(cv == mc, ci, big), axis=0, keepdims=True)
        accept = (mc < acc_v) | ((mc == acc_v) & (ixc < acc_i))
        acc_v = jnp.where(accept, mc.astype(jnp.bfloat16).astype(jnp.float32),
                          acc_v)
        acc_i = jnp.where(accept, ixc, acc_i)
    idx_ref[0, 0, :] = acc_i[0]


def _nearest_code(flat, sum_f, sum_e, emb):
    out = pl.pallas_call(
        _argmin_body,
        grid=(_NT,),
        in_specs=[
            pl.BlockSpec((_T, _EMBED_DIM), lambda i: (i, 0)),
            pl.BlockSpec((1, _T), lambda i: (0, i)),
            pl.BlockSpec((_ROWS, 8, 1), lambda i: (0, 0, 0)),
            pl.BlockSpec((_N_EMBED, _EMBED_DIM), lambda i: (0, 0)),
        ],
        out_specs=pl.BlockSpec((1, 1, _T), lambda i: (i, 0, 0)),
        out_shape=jax.ShapeDtypeStruct((_NT, 1, _T), jnp.int32),
        scratch_shapes=[pltpu.VMEM((_ROWS, 8, _T), jnp.float32)],
    )(flat, sum_f, sum_e, emb)
    return out.reshape(-1)


def _make_sc_gather():
    info = plsc.get_sparse_core_info()
    nw = info.num_cores * info.num_subcores
    b_per_w = _N_EMBED // nw
    mesh = plsc.VectorSubcoreMesh(core_axis_name="c", subcore_axis_name="s")

    @functools.partial(
        pl.kernel, mesh=mesh,
        compiler_params=pltpu.CompilerParams(use_tc_tiling_on_sc=False),
        out_type=jax.ShapeDtypeStruct((8192, _EMBED_DIM), jnp.float32),
        scratch_types=[
            pltpu.VMEM((b_per_w,), jnp.int32),
            pltpu.VMEM((b_per_w, _EMBED_DIM), jnp.float32),
            pltpu.SemaphoreType.DMA,
        ],
    )
    def gather_kernel(table_hbm, idx_hbm, out_hbm, idx_v, rows_v, sem):
        wid = lax.axis_index("s") * info.num_cores + lax.axis_index("c")
        base = wid * b_per_w
        pltpu.sync_copy(idx_hbm.at[pl.ds(base, b_per_w)], idx_v)
        pltpu.async_copy(table_hbm.at[idx_v], rows_v, sem).wait()
        pltpu.sync_copy(rows_v, out_hbm.at[pl.ds(base, b_per_w)])

    return gather_kernel


_sc_gather = _make_sc_gather()


def kernel(hidden_states, emb_weights):
    b, c, h, w = hidden_states.shape
    hs = jnp.transpose(hidden_states, (0, 2, 3, 1))
    flat = hs.reshape((-1, _EMBED_DIM))
    sum_f = jnp.sum(flat ** 2, axis=1)[None, :]
    sum_e = jnp.sum(emb_weights ** 2, axis=1).reshape(_ROWS, 8, 1)

    indices = _nearest_code(flat, sum_f, sum_e, emb_weights)
    z_q_flat = _sc_gather(emb_weights, indices)

    z_q = z_q_flat.reshape((b, h, w, c))
    z_q = jnp.transpose(z_q, (0, 3, 1, 2))
    return (z_q, indices.reshape(b, -1))


# speedup vs baseline: 6.2867x; 6.2867x over previous
import functools

import jax
import jax.numpy as jnp
from jax import lax
from jax.experimental import pallas as pl
from jax.experimental.pallas import tpu as pltpu
from jax.experimental.pallas import tpu_sc as plsc

_N_EMBED = 8192
_EMBED_DIM = 32
_T = 512
_NT = 8192 // _T
_CHUNK = 2048
_ROWS = _N_EMBED // 8


def _argmin_body(flat_ref, sumf_ref, sume_ref, emb_ref, idx_ref, mt_ref):
    f = flat_ref[...]
    m2 = lax.dot_general(emb_ref[...] * 2.0, f, (((1,), (1,)), ((), ())),
                         preferred_element_type=jnp.float32)
    mt_ref[...] = m2.reshape(_ROWS, 8, _T)

    sumf = sumf_ref[...]
    iota0 = lax.broadcasted_iota(jnp.int32, (8, _T), 0)
    big = jnp.int32(2**31 - 1)

    acc_v = jnp.full((1, _T), jnp.inf, jnp.float32)
    acc_i = jnp.zeros((1, _T), jnp.int32)
    rows_per_chunk = _CHUNK // 8
    unroll = 16
    for c in range(_N_EMBED // _CHUNK):
        def body(u, carry, c=c):
            cv, ci = carry
            for k in range(unroll):
                rg = c * rows_per_chunk + u * unroll + k
                mrow = mt_ref[rg]
                d = (sumf + sume_ref[rg]) - mrow
                lt = d < cv
                cv = jnp.where(lt, d, cv)
                ci = jnp.where(lt, iota0 + rg * 8, ci)
            return cv, ci
        cv, ci = lax.fori_loop(
            0, rows_per_chunk // unroll, body,
            (jnp.full((8, _T), jnp.inf, jnp.float32),
             jnp.zeros((8, _T), jnp.int32)))
        mc = jnp.min(cv, axis=0, keepdims=True)
        ixc = jnp.min(jnp.where(cv == mc, ci, big), axis=0, keepdims=True)
        accept = (mc < acc_v) | ((mc == acc_v) & (ixc < acc_i))
        acc_v = jnp.where(accept, mc.astype(jnp.bfloat16).astype(jnp.float32),
                          acc_v)
        acc_i = jnp.where(accept, ixc, acc_i)
    idx_ref[0, 0, :] = acc_i[0]


def _nearest_code(flat, sum_f, sum_e, emb):
    out = pl.pallas_call(
        _argmin_body,
        grid=(_NT,),
        in_specs=[
            pl.BlockSpec((_T, _EMBED_DIM), lambda i: (i, 0)),
            pl.BlockSpec((1, _T), lambda i: (0, i)),
            pl.BlockSpec((_ROWS, 8, 1), lambda i: (0, 0, 0)),
            pl.BlockSpec((_N_EMBED, _EMBED_DIM), lambda i: (0, 0)),
        ],
        out_specs=pl.BlockSpec((1, 1, _T), lambda i: (i, 0, 0)),
        out_shape=jax.ShapeDtypeStruct((_NT, 1, _T), jnp.int32),
        scratch_shapes=[pltpu.VMEM((_ROWS, 8, _T), jnp.float32)],
    )(flat, sum_f, sum_e, emb)
    return out.reshape(-1)


def _make_sc_gather():
    info = plsc.get_sparse_core_info()
    nw = info.num_cores * info.num_subcores
    b_per_w = _N_EMBED // nw
    mesh = plsc.VectorSubcoreMesh(core_axis_name="c", subcore_axis_name="s")

    @functools.partial(
        pl.kernel, mesh=mesh,
        compiler_params=pltpu.CompilerParams(use_tc_tiling_on_sc=False),
        out_type=jax.ShapeDtypeStruct((8192, _EMBED_DIM), jnp.float32),
        scratch_types=[
            pltpu.VMEM((b_per_w,), jnp.int32),
            pltpu.VMEM((b_per_w, _EMBED_DIM), jnp.float32),
            pltpu.SemaphoreType.DMA,
        ],
    )
    def gather_kernel(table_hbm, idx_hbm, out_hbm, idx_v, rows_v, sem):
        wid = lax.axis_index("s") * info.num_cores + lax.axis_index("c")
        base = wid * b_per_w
        pltpu.sync_copy(idx_hbm.at[pl.ds(base, b_per_w)], idx_v)
        pltpu.async_copy(table_hbm.at[idx_v], rows_v, sem).wait()
        pltpu.sync_copy(rows_v, out_hbm.at[pl.ds(base, b_per_w)])

    return gather_kernel


_sc_gather = _make_sc_gather()


def kernel(hidden_states, emb_weights):
    b, c, h, w = hidden_states.shape
    hs = jnp.transpose(hidden_states, (0, 2, 3, 1))
    flat = hs.reshape((-1, _EMBED_DIM))
    sum_f = jnp.sum(flat ** 2, axis=1)[None, :]
    sum_e = jnp.sum(emb_weights ** 2, axis=1).reshape(_ROWS, 8, 1)

    indices = _nearest_code(flat, sum_f, sum_e, emb_weights)
    z_q_flat = _sc_gather(emb_weights, indices)

    z_q = z_q_flat.reshape((b, h, w, c))
    z_q = jnp.transpose(z_q, (0, 3, 1, 2))
    return (z_q, indices.reshape(b, -1))


# speedup vs baseline: 9.2756x; 1.4754x over previous
import functools

import jax
import jax.numpy as jnp
from jax import lax
from jax.experimental import pallas as pl
from jax.experimental.pallas import tpu as pltpu
from jax.experimental.pallas import tpu_sc as plsc

_N_EMBED = 8192
_EMBED_DIM = 32
_T = 512
_NT = 8192 // _T
_CHUNK = 2048
_ROWS = _N_EMBED // 8


def _argmin_body(flat_ref, sumf_ref, sume_ref, jcol_ref, emb_ref, idx_ref):
    f = flat_ref[...]
    m2 = lax.dot_general(emb_ref[...] * 2.0, f, (((1,), (1,)), ((), ())),
                         preferred_element_type=jnp.float32)
    d = (sumf_ref[...] + sume_ref[...]) - m2

    acc_v = jnp.full((1, _T), jnp.inf, jnp.float32)
    acc_i = jnp.zeros((1, _T), jnp.int32)
    for c in range(_N_EMBED // _CHUNK):
        dc = d[c * _CHUNK:(c + 1) * _CHUNK, :]
        mc = jnp.min(dc, axis=0, keepdims=True)
        jc = jcol_ref[c * _CHUNK:(c + 1) * _CHUNK, :]
        ixf = jnp.min(jnp.where(dc == mc, jc, jnp.inf), axis=0, keepdims=True)
        ixc = ixf.astype(jnp.int32)
        accept = (mc < acc_v) | ((mc == acc_v) & (ixc < acc_i))
        acc_v = jnp.where(accept, mc.astype(jnp.bfloat16).astype(jnp.float32),
                          acc_v)
        acc_i = jnp.where(accept, ixc, acc_i)
    idx_ref[0, 0, :] = acc_i[0]


def _nearest_code(flat, sum_f, sum_e, jcol, emb):
    out = pl.pallas_call(
        _argmin_body,
        grid=(_NT,),
        in_specs=[
            pl.BlockSpec((_T, _EMBED_DIM), lambda i: (i, 0)),
            pl.BlockSpec((1, _T), lambda i: (0, i)),
            pl.BlockSpec((_N_EMBED, 1), lambda i: (0, 0)),
            pl.BlockSpec((_N_EMBED, 1), lambda i: (0, 0)),
            pl.BlockSpec((_N_EMBED, _EMBED_DIM), lambda i: (0, 0)),
        ],
        out_specs=pl.BlockSpec((1, 1, _T), lambda i: (i, 0, 0)),
        out_shape=jax.ShapeDtypeStruct((_NT, 1, _T), jnp.int32),
    )(flat, sum_f, sum_e, jcol, emb)
    return out.reshape(-1)


def _make_sc_gather():
    info = plsc.get_sparse_core_info()
    nw = info.num_cores * info.num_subcores
    b_per_w = _N_EMBED // nw
    mesh = plsc.VectorSubcoreMesh(core_axis_name="c", subcore_axis_name="s")

    @functools.partial(
        pl.kernel, mesh=mesh,
        compiler_params=pltpu.CompilerParams(use_tc_tiling_on_sc=False),
        out_type=jax.ShapeDtypeStruct((8192, _EMBED_DIM), jnp.float32),
        scratch_types=[
            pltpu.VMEM((b_per_w,), jnp.int32),
            pltpu.VMEM((b_per_w, _EMBED_DIM), jnp.float32),
            pltpu.SemaphoreType.DMA,
        ],
    )
    def gather_kernel(table_hbm, idx_hbm, out_hbm, idx_v, rows_v, sem):
        wid = lax.axis_index("s") * info.num_cores + lax.axis_index("c")
        base = wid * b_per_w
        pltpu.sync_copy(idx_hbm.at[pl.ds(base, b_per_w)], idx_v)
        pltpu.async_copy(table_hbm.at[idx_v], rows_v, sem).wait()
        pltpu.sync_copy(rows_v, out_hbm.at[pl.ds(base, b_per_w)])

    return gather_kernel


_sc_gather = _make_sc_gather()


def kernel(hidden_states, emb_weights):
    b, c, h, w = hidden_states.shape
    hs = jnp.transpose(hidden_states, (0, 2, 3, 1))
    flat = hs.reshape((-1, _EMBED_DIM))
    sum_f = jnp.sum(flat ** 2, axis=1)[None, :]
    sum_e = jnp.sum(emb_weights ** 2, axis=1)[:, None]
    jcol = jnp.arange(_N_EMBED, dtype=jnp.float32)[:, None]

    indices = _nearest_code(flat, sum_f, sum_e, jcol, emb_weights)
    z_q_flat = _sc_gather(emb_weights, indices)

    z_q = z_q_flat.reshape((b, h, w, c))
    z_q = jnp.transpose(z_q, (0, 3, 1, 2))
    return (z_q, indices.reshape(b, -1))
